# baseline (speedup 1.0000x reference)
import jax
import jax.numpy as jnp
from jax import lax
from jax.experimental import pallas as pl

B, T, N, D_IN = 4, 24, 8192, 32
HID, NH, P, D_OUT = 16, 4, 6, 32
G = T // P
F = NH * HID
NB = 512
AC = NH * P * G * G


def _tb_kernel(x_ref, w2dT_ref, wsrcT_ref, wdstT_ref, woutT_ref, bout_ref,
               rep_ref, perm_ref, out_ref, attn_ref):
    x = x_ref[0]
    x2 = x.reshape(T * NB, D_IN)

    def matT(w, v):
        return lax.dot_general(w, v, (((1,), (1,)), ((), ())),
                               preferred_element_type=jnp.float32)

    hT = matT(w2dT_ref[...], x2)
    esT = matT(wsrcT_ref[...], x2)
    edT = matT(wdstT_ref[...], x2)

    a_chunks = []
    for p in range(P):
        base = p * G * NB
        src = [esT[:, base + i * NB: base + (i + 1) * NB] for i in range(G)]
        dst = [edT[:, base + j * NB: base + (j + 1) * NB] for j in range(G)]
        for i in range(G):
            e_row = []
            for j in range(G):
                e = src[i] + dst[j]
                e_row.append(jnp.where(e >= 0.0, e, 0.2 * e))
            m = jnp.maximum(jnp.maximum(e_row[0], e_row[1]),
                            jnp.maximum(e_row[2], e_row[3]))
            ex = [jnp.exp(e - m) for e in e_row]
            inv = 1.0 / (ex[0] + ex[1] + ex[2] + ex[3])
            a_chunks.append(jnp.concatenate([exj * inv for exj in ex], axis=1))

    a_cat = jnp.concatenate(a_chunks, axis=1)
    arep = jnp.dot(rep_ref[...], a_cat, preferred_element_type=jnp.float32)
    o_chunks = []
    for p in range(P):
        hslab = hT[:, p * G * NB:(p + 1) * G * NB]
        for i in range(G):
            c = arep[:, (p * G + i) * G * NB:(p * G + i + 1) * G * NB] * hslab
            o_chunks.append(c[:, 0:NB] + c[:, NB:2 * NB]
                            + c[:, 2 * NB:3 * NB] + c[:, 3 * NB:4 * NB])

    oT = jnp.concatenate(o_chunks, axis=1)
    zT = jnp.dot(woutT_ref[...], oT, preferred_element_type=jnp.float32)
    zT = zT + bout_ref[...]
    zT = jnp.where(zT > 0.0, zT, jnp.exp(zT) - 1.0)
    out_ref[0] = (x2 + zT.T).reshape(T, NB, D_IN)

    a0 = a_cat.reshape(NH * P * G * G, NB)
    attn_ref[...] = lax.dot_general(a0, perm_ref[...], (((0,), (0,)), ((), ())),
                                    preferred_element_type=jnp.float32)


def kernel(input, covariate, W, a_src, a_dst, W_out, b_out):
    del covariate
    w2dT = jnp.transpose(W, (0, 2, 1)).reshape(F, D_IN)
    wsrcT = jnp.einsum('ndh,nh->nd', W, a_src)
    wdstT = jnp.einsum('ndh,nh->nd', W, a_dst)
    woutT = W_out.T
    bout = b_out.reshape(D_OUT, 1)
    rep = (jnp.arange(F)[:, None] // HID
           == jnp.arange(NH)[None, :]).astype(jnp.float32)
    d = jnp.arange(AC)
    perm = (d[:, None] == d[None, :]).astype(jnp.float32)

    nblk = N // NB
    out, attn2 = pl.pallas_call(
        _tb_kernel,
        grid=(B, nblk),
        in_specs=[
            pl.BlockSpec((1, T, NB, D_IN), lambda b, k: (b, 0, k, 0)),
            pl.BlockSpec((F, D_IN), lambda b, k: (0, 0)),
            pl.BlockSpec((NH, D_IN), lambda b, k: (0, 0)),
            pl.BlockSpec((NH, D_IN), lambda b, k: (0, 0)),
            pl.BlockSpec((D_OUT, F), lambda b, k: (0, 0)),
            pl.BlockSpec((D_OUT, 1), lambda b, k: (0, 0)),
            pl.BlockSpec((F, NH), lambda b, k: (0, 0)),
            pl.BlockSpec((AC, AC), lambda b, k: (0, 0)),
        ],
        out_specs=[
            pl.BlockSpec((1, T, NB, D_IN), lambda b, k: (b, 0, k, 0)),
            pl.BlockSpec((NB, AC), lambda b, k: (b * nblk + k, 0)),
        ],
        out_shape=[
            jax.ShapeDtypeStruct((B, T, N, D_IN), jnp.float32),
            jax.ShapeDtypeStruct((B * N, AC), jnp.float32),
        ],
    )(input, w2dT, wsrcT, wdstT, woutT, bout, rep, perm)

    return (out, attn2.reshape(B * N, NH, P, G, G))
